# Initial kernel scaffold; baseline (speedup 1.0000x reference)
import jax
import jax.numpy as jnp
from jax.experimental import pallas as pl


def kernel(x, edge_index, pos_edge_index, W1l, W1r, b1, W2l, W2r, b2):
    raise NotImplementedError("write your pallas kernel here")


# speedup vs baseline: 1.6641x; 1.6641x over previous
import functools

import jax
import jax.numpy as jnp
from jax import lax
from jax.experimental import pallas as pl
from jax.experimental.pallas import tpu as pltpu
from jax.experimental.pallas import tpu_sc as plsc

N = 10000
E = 160000
DIN = 256
DH = 512
DOUT = 256

FW = 128
NC = 2
NS = 16
NW = NC * NS
NP = 10112
CH = 100
EPW = E // NW
NCH = EPW // CH
RPT = NP // NS

CH2 = 50
EPW2 = E // NW
NCH2 = EPW2 // CH2

_MESH = plsc.VectorSubcoreMesh(core_axis_name="c", subcore_axis_name="s")


def _make_sc_agg(nparts, with_deg):
    out_type = [jax.ShapeDtypeStruct((nparts, NC, NP, FW), jnp.float32)]
    if with_deg:
        out_type.append(jax.ShapeDtypeStruct((NC * NP,), jnp.float32))

    scratch = [
        pltpu.VMEM_SHARED((NP, FW), jnp.float32),
        pltpu.VMEM((NCH, CH), jnp.int32),
        pltpu.VMEM((NCH, CH), jnp.int32),
        pltpu.VMEM((CH, FW), jnp.float32),
        pltpu.VMEM((CH, FW), jnp.float32),
        pltpu.SemaphoreType.DMA,
        pltpu.SemaphoreType.DMA,
        pltpu.SemaphoreType.DMA,
        pltpu.SemaphoreType.DMA,
    ]
    if with_deg:
        scratch += [
            pltpu.VMEM_SHARED((NP,), jnp.float32),
            pltpu.VMEM((CH,), jnp.float32),
            pltpu.VMEM((RPT,), jnp.float32),
            pltpu.SemaphoreType.DMA,
            pltpu.SemaphoreType.DMA,
        ]

    def body(*refs):
        if with_deg:
            (table, srcq, dstq, zeros_h, zeros1_h, ones_h,
             agg_out, deg_out,
             acc, sidx, didx, r0, r1, g0, g1, s0, s1,
             dacc, ones_v, vbuf, d0, d1) = refs
        else:
            (table, srcq, dstq, zeros_h,
             agg_out,
             acc, sidx, didx, r0, r1, g0, g1, s0, s1) = refs

        c = lax.axis_index("c")
        s = lax.axis_index("s")
        rows = [r0, r1]
        gsem = [g0, g1]
        ssem = [s0, s1]
        if with_deg:
            dsem = [d0, d1]
            pltpu.sync_copy(ones_h, ones_v)

        pltpu.sync_copy(dstq.at[c, s], didx)

        for p in range(nparts):
            pltpu.sync_copy(zeros_h.at[pl.ds(s * RPT, RPT)],
                            acc.at[pl.ds(s * RPT, RPT)])
            if with_deg and p == 0:
                pltpu.sync_copy(zeros1_h.at[pl.ds(s * RPT, RPT)], vbuf)
                pltpu.sync_copy(vbuf, dacc.at[pl.ds(s * RPT, RPT)])
            plsc.subcore_barrier()

            pltpu.sync_copy(srcq.at[p, c, s], sidx)

            def fire_g(i, b):
                pltpu.async_copy(table.at[sidx.at[i]], rows[b], gsem[b])

            def wait_g(i, b):
                pltpu.make_async_copy(table.at[sidx.at[i]], rows[b],
                                      gsem[b]).wait()

            def fire_s(i, b):
                pltpu.async_copy(rows[b], acc.at[didx.at[i]], ssem[b],
                                 add=True)

            def wait_s(i, b):
                pltpu.make_async_copy(rows[b], acc.at[didx.at[i]],
                                      ssem[b]).wait()

            def fire_d(i, b):
                if with_deg and p == 0:
                    pltpu.async_copy(ones_v, dacc.at[didx.at[i]],
                                     dsem[b], add=True)

            def wait_d(i, b):
                if with_deg and p == 0:
                    pltpu.make_async_copy(ones_v, dacc.at[didx.at[i]],
                                          dsem[b]).wait()

            fire_g(0, 0)
            wait_g(0, 0)
            fire_s(0, 0)
            fire_d(0, 0)
            fire_g(1, 1)

            def grp(g, carry):
                for j in range(2):
                    i = 2 * g + 1 + j
                    b = (1 + j) % 2
                    wait_g(i, b)
                    fire_s(i, b)
                    fire_d(i, b)
                    wait_s(i - 1, 1 - b)
                    wait_d(i - 1, 1 - b)

                    @pl.when(i + 1 < NCH)
                    def _():
                        fire_g(i + 1, 1 - b)
                return carry

            lax.fori_loop(0, (NCH - 2) // 2, grp, None)
            i = NCH - 1
            wait_g(i, i % 2)
            fire_s(i, i % 2)
            fire_d(i, i % 2)
            wait_s(i - 1, (i - 1) % 2)
            wait_d(i - 1, (i - 1) % 2)
            wait_s(i, i % 2)
            wait_d(i, i % 2)

            plsc.subcore_barrier()

            pltpu.sync_copy(acc.at[pl.ds(s * RPT, RPT)],
                            agg_out.at[p, c, pl.ds(s * RPT, RPT)])
            if with_deg and p == 0:
                pltpu.sync_copy(dacc.at[pl.ds(s * RPT, RPT)], vbuf)
                pltpu.sync_copy(vbuf, deg_out.at[pl.ds(c * NP + s * RPT, RPT)])

    return functools.partial(
        pl.kernel, body, out_type=out_type, mesh=_MESH,
        scratch_types=scratch)()


def _sc_decode(z, ps, pd):
    scratch = [
        pltpu.VMEM((NCH2, CH2), jnp.int32),
        pltpu.VMEM((NCH2, CH2), jnp.int32),
        pltpu.VMEM((CH2, DOUT), jnp.float32),
        pltpu.VMEM((CH2, DOUT), jnp.float32),
        pltpu.VMEM((CH2, DOUT), jnp.float32),
        pltpu.VMEM((CH2, DOUT), jnp.float32),
        pltpu.VMEM((NCH2, 64), jnp.float32),
        pltpu.SemaphoreType.DMA,
        pltpu.SemaphoreType.DMA,
        pltpu.SemaphoreType.DMA,
        pltpu.SemaphoreType.DMA,
    ]

    def body(z_h, ps_h, pd_h, out_h,
             psb, pdb, zs0, zs1, zd0, zd1, outv, a0, a1, b0, b1):
        c = lax.axis_index("c")
        s = lax.axis_index("s")
        w = s * NC + c
        zs = [zs0, zs1]
        zd = [zd0, zd1]
        asem = [a0, a1]
        bsem = [b0, b1]

        pltpu.sync_copy(ps_h.at[w], psb)
        pltpu.sync_copy(pd_h.at[w], pdb)

        def fire(i, b):
            pltpu.async_copy(z_h.at[psb.at[i]], zs[b], asem[b])
            pltpu.async_copy(z_h.at[pdb.at[i]], zd[b], bsem[b])

        def wait(i, b):
            pltpu.make_async_copy(z_h.at[psb.at[i]], zs[b], asem[b]).wait()
            pltpu.make_async_copy(z_h.at[pdb.at[i]], zd[b], bsem[b]).wait()

        ng = (CH2 + 15) // 16
        iota = lax.broadcasted_iota(jnp.int32, (16,), 0)
        eidx = [jnp.minimum(iota + g * 16, CH2 - 1) for g in range(ng)]

        def compute(i, b):
            def feat(f, accs):
                fi = jnp.full((16,), f, dtype=jnp.int32)
                out = []
                for g in range(ng):
                    vs = plsc.load_gather(zs[b], [eidx[g], fi])
                    vd = plsc.load_gather(zd[b], [eidx[g], fi])
                    out.append(accs[g] + vs * vd)
                return tuple(out)

            z16 = jnp.zeros((16,), jnp.float32)
            accs = lax.fori_loop(0, DOUT, feat, (z16,) * ng)
            for g in range(ng):
                outv[i, pl.ds(g * 16, 16)] = accs[g]

        fire(0, 0)

        def grp(g, carry):
            wait(2 * g, 0)
            fire(2 * g + 1, 1)
            compute(2 * g, 0)
            wait(2 * g + 1, 1)

            @pl.when(g < NCH2 // 2 - 1)
            def _():
                fire(2 * g + 2, 0)
            compute(2 * g + 1, 1)
            return carry

        lax.fori_loop(0, NCH2 // 2, grp, None)
        pltpu.sync_copy(outv, out_h.at[w])

    return pl.kernel(
        body,
        out_type=jax.ShapeDtypeStruct((NW, NCH2, 64), jnp.float32),
        mesh=_MESH,
        compiler_params=pltpu.CompilerParams(use_tc_tiling_on_sc=False,
                                             needs_layout_passes=False),
        scratch_types=scratch)(z, ps, pd)


_B = 1000


def _tc_layer1(a2, d0, d1, x, W1l, W1r, b1):
    def body(a_ref, d0_ref, d1_ref, x_ref, wl_ref, wr_ref, b_ref, out_ref):
        recip = 1.0 / jnp.maximum(d0_ref[...] + d1_ref[...], 1.0)
        a = jnp.concatenate(
            [a_ref[qq, 0] + a_ref[qq, 1] for qq in range(2)], axis=1) * recip
        h = jnp.dot(a, wl_ref[...], preferred_element_type=jnp.float32)
        h = h + jnp.dot(x_ref[...], wr_ref[...],
                        preferred_element_type=jnp.float32)
        h = jnp.maximum(h + b_ref[...], 0.0)
        for qq in range(4):
            out_ref[qq] = h[:, qq * FW:(qq + 1) * FW]

    return pl.pallas_call(
        body,
        grid=(N // _B,),
        in_specs=[
            pl.BlockSpec((2, NC, _B, FW), lambda i: (0, 0, i, 0)),
            pl.BlockSpec((_B, 1), lambda i: (i, 0)),
            pl.BlockSpec((_B, 1), lambda i: (i, 0)),
            pl.BlockSpec((_B, DIN), lambda i: (i, 0)),
            pl.BlockSpec((DIN, DH), lambda i: (0, 0)),
            pl.BlockSpec((DIN, DH), lambda i: (0, 0)),
            pl.BlockSpec((1, DH), lambda i: (0, 0)),
        ],
        out_specs=pl.BlockSpec((4, _B, FW), lambda i: (0, i, 0)),
        out_shape=jax.ShapeDtypeStruct((4, NP, FW), jnp.float32),
    )(a2, d0, d1, x, W1l, W1r, b1.reshape(1, DH))


def _tc_layer2(a4, d0, d1, h4, W2l, W2r, b2):
    def body(a_ref, d0_ref, d1_ref, h_ref, wl_ref, wr_ref, b_ref, out_ref):
        recip = 1.0 / jnp.maximum(d0_ref[...] + d1_ref[...], 1.0)
        a = jnp.concatenate(
            [a_ref[qq, 0] + a_ref[qq, 1] for qq in range(4)], axis=1) * recip
        hm = jnp.concatenate([h_ref[qq] for qq in range(4)], axis=1)
        z = jnp.dot(a, wl_ref[...], preferred_element_type=jnp.float32)
        z = z + jnp.dot(hm, wr_ref[...], preferred_element_type=jnp.float32)
        out_ref[...] = z + b_ref[...]

    return pl.pallas_call(
        body,
        grid=(N // _B,),
        in_specs=[
            pl.BlockSpec((4, NC, _B, FW), lambda i: (0, 0, i, 0)),
            pl.BlockSpec((_B, 1), lambda i: (i, 0)),
            pl.BlockSpec((_B, 1), lambda i: (i, 0)),
            pl.BlockSpec((4, _B, FW), lambda i: (0, i, 0)),
            pl.BlockSpec((DH, DOUT), lambda i: (0, 0)),
            pl.BlockSpec((DH, DOUT), lambda i: (0, 0)),
            pl.BlockSpec((1, DOUT), lambda i: (0, 0)),
        ],
        out_specs=pl.BlockSpec((_B, DOUT), lambda i: (i, 0)),
        out_shape=jax.ShapeDtypeStruct((N, DOUT), jnp.float32),
    )(a4, d0, d1, h4, W2l, W2r, b2.reshape(1, DOUT))


def kernel(x, edge_index, pos_edge_index, W1l, W1r, b1, W2l, W2r, b2):
    src = edge_index[0]
    dst = edge_index[1]

    off2 = (jnp.arange(2, dtype=jnp.int32) * NP)[:, None]
    off4 = (jnp.arange(4, dtype=jnp.int32) * NP)[:, None]
    src2 = (src[None, :] + off2).reshape(2, NC, NS, NCH, CH)
    src4 = (src[None, :] + off4).reshape(4, NC, NS, NCH, CH)
    dst4 = dst.reshape(NC, NS, NCH, CH)

    xp = jnp.concatenate([x, jnp.zeros((NP - N, DIN), jnp.float32)])
    x2 = xp.reshape(NP, 2, FW).transpose(1, 0, 2).reshape(2 * NP, FW)
    zeros_h = jnp.zeros((NP, FW), jnp.float32)
    zeros1 = jnp.zeros((NP,), jnp.float32)
    ones_h = jnp.ones((CH,), jnp.float32)

    agg1, deg = _make_sc_agg(2, True)(
        x2, src2, dst4, zeros_h, zeros1, ones_h)
    d0 = deg[:NP].reshape(NP, 1)
    d1 = deg[NP:].reshape(NP, 1)
    h4 = _tc_layer1(agg1, d0, d1, x, W1l, W1r, b1)
    agg2, = _make_sc_agg(4, False)(h4.reshape(4 * NP, FW), src4, dst4, zeros_h)
    z = _tc_layer2(agg2, d0, d1, h4, W2l, W2r, b2)

    ps = pos_edge_index[0].reshape(NW, NCH2, CH2)
    pd = pos_edge_index[1].reshape(NW, NCH2, CH2)
    logits = _sc_decode(z, ps, pd)
    return logits[:, :, :CH2].reshape(E)


# speedup vs baseline: 1.7663x; 1.0614x over previous
import functools

import jax
import jax.numpy as jnp
from jax import lax
from jax.experimental import pallas as pl
from jax.experimental.pallas import tpu as pltpu
from jax.experimental.pallas import tpu_sc as plsc

N = 10000
E = 160000
DIN = 256
DH = 512
DOUT = 256

FW = 128
NC = 2
NS = 16
NW = NC * NS
NP = 10112
CH = 100
EPW = E // NW
NCH = EPW // CH
RPT = NP // NS

CH2 = 50
EPW2 = E // NW
NCH2 = EPW2 // CH2

_MESH = plsc.VectorSubcoreMesh(core_axis_name="c", subcore_axis_name="s")


def _make_sc_agg(nparts, with_deg):
    out_type = [jax.ShapeDtypeStruct((nparts, NC, NP, FW), jnp.float32)]
    if with_deg:
        out_type.append(jax.ShapeDtypeStruct((NC * NP,), jnp.float32))

    scratch = [
        pltpu.VMEM_SHARED((NP, FW), jnp.float32),
        pltpu.VMEM((NCH, CH), jnp.int32),
        pltpu.VMEM((NCH, CH), jnp.int32),
        pltpu.VMEM((CH, FW), jnp.float32),
        pltpu.VMEM((CH, FW), jnp.float32),
        pltpu.SemaphoreType.DMA,
        pltpu.SemaphoreType.DMA,
        pltpu.SemaphoreType.DMA,
        pltpu.SemaphoreType.DMA,
    ]
    if with_deg:
        scratch += [
            pltpu.VMEM_SHARED((NP,), jnp.float32),
            pltpu.VMEM((CH,), jnp.float32),
            pltpu.VMEM((RPT,), jnp.float32),
            pltpu.SemaphoreType.DMA,
            pltpu.SemaphoreType.DMA,
        ]

    def body(*refs):
        if with_deg:
            (table, srcq, dstq, zeros_h, zeros1_h, ones_h,
             agg_out, deg_out,
             acc, sidx, didx, r0, r1, g0, g1, s0, s1,
             dacc, ones_v, vbuf, d0, d1) = refs
        else:
            (table, srcq, dstq, zeros_h,
             agg_out,
             acc, sidx, didx, r0, r1, g0, g1, s0, s1) = refs

        c = lax.axis_index("c")
        s = lax.axis_index("s")
        rows = [r0, r1]
        gsem = [g0, g1]
        ssem = [s0, s1]
        if with_deg:
            dsem = [d0, d1]
            pltpu.sync_copy(ones_h, ones_v)

        pltpu.sync_copy(dstq.at[c, s], didx)

        for p in range(nparts):
            pltpu.sync_copy(zeros_h.at[pl.ds(s * RPT, RPT)],
                            acc.at[pl.ds(s * RPT, RPT)])
            if with_deg and p == 0:
                pltpu.sync_copy(zeros1_h.at[pl.ds(s * RPT, RPT)], vbuf)
                pltpu.sync_copy(vbuf, dacc.at[pl.ds(s * RPT, RPT)])
            plsc.subcore_barrier()

            pltpu.sync_copy(srcq.at[p, c, s], sidx)

            def fire_g(i, b):
                pltpu.async_copy(table.at[sidx.at[i]], rows[b], gsem[b])

            def wait_g(i, b):
                pltpu.make_async_copy(table.at[sidx.at[i]], rows[b],
                                      gsem[b]).wait()

            def fire_s(i, b):
                pltpu.async_copy(rows[b], acc.at[didx.at[i]], ssem[b],
                                 add=True)

            def wait_s(i, b):
                pltpu.make_async_copy(rows[b], acc.at[didx.at[i]],
                                      ssem[b]).wait()

            def fire_d(i, b):
                if with_deg and p == 0:
                    pltpu.async_copy(ones_v, dacc.at[didx.at[i]],
                                     dsem[b], add=True)

            def wait_d(i, b):
                if with_deg and p == 0:
                    pltpu.make_async_copy(ones_v, dacc.at[didx.at[i]],
                                          dsem[b]).wait()

            fire_g(0, 0)
            wait_g(0, 0)
            fire_s(0, 0)
            fire_d(0, 0)
            fire_g(1, 1)

            def grp(g, carry):
                for j in range(2):
                    i = 2 * g + 1 + j
                    b = (1 + j) % 2
                    wait_g(i, b)
                    fire_s(i, b)
                    fire_d(i, b)
                    wait_s(i - 1, 1 - b)
                    wait_d(i - 1, 1 - b)

                    @pl.when(i + 1 < NCH)
                    def _():
                        fire_g(i + 1, 1 - b)
                return carry

            lax.fori_loop(0, (NCH - 2) // 2, grp, None)
            i = NCH - 1
            wait_g(i, i % 2)
            fire_s(i, i % 2)
            fire_d(i, i % 2)
            wait_s(i - 1, (i - 1) % 2)
            wait_d(i - 1, (i - 1) % 2)
            wait_s(i, i % 2)
            wait_d(i, i % 2)

            plsc.subcore_barrier()

            pltpu.sync_copy(acc.at[pl.ds(s * RPT, RPT)],
                            agg_out.at[p, c, pl.ds(s * RPT, RPT)])
            if with_deg and p == 0:
                pltpu.sync_copy(dacc.at[pl.ds(s * RPT, RPT)], vbuf)
                pltpu.sync_copy(vbuf, deg_out.at[pl.ds(c * NP + s * RPT, RPT)])

    return functools.partial(
        pl.kernel, body, out_type=out_type, mesh=_MESH,
        scratch_types=scratch)()


def _sc_decode(z, ps, pd):
    scratch = [
        pltpu.VMEM((NCH2, CH2), jnp.int32),
        pltpu.VMEM((NCH2, CH2), jnp.int32),
        pltpu.VMEM((CH2, DOUT), jnp.float32),
        pltpu.VMEM((CH2, DOUT), jnp.float32),
        pltpu.VMEM((CH2, DOUT), jnp.float32),
        pltpu.VMEM((CH2, DOUT), jnp.float32),
        pltpu.VMEM((NCH2, 64), jnp.float32),
        pltpu.SemaphoreType.DMA,
        pltpu.SemaphoreType.DMA,
        pltpu.SemaphoreType.DMA,
        pltpu.SemaphoreType.DMA,
    ]

    def body(z_h, ps_h, pd_h, out_h,
             psb, pdb, zs0, zs1, zd0, zd1, outv, a0, a1, b0, b1):
        c = lax.axis_index("c")
        s = lax.axis_index("s")
        w = s * NC + c
        zs = [zs0, zs1]
        zd = [zd0, zd1]
        asem = [a0, a1]
        bsem = [b0, b1]

        pltpu.sync_copy(ps_h.at[w], psb)
        pltpu.sync_copy(pd_h.at[w], pdb)

        def fire(i, b):
            pltpu.async_copy(z_h.at[psb.at[i]], zs[b], asem[b])
            pltpu.async_copy(z_h.at[pdb.at[i]], zd[b], bsem[b])

        def wait(i, b):
            pltpu.make_async_copy(z_h.at[psb.at[i]], zs[b], asem[b]).wait()
            pltpu.make_async_copy(z_h.at[pdb.at[i]], zd[b], bsem[b]).wait()

        ng = (CH2 + 15) // 16
        iota = lax.broadcasted_iota(jnp.int32, (16,), 0)
        eidx = [jnp.minimum(iota + g * 16, CH2 - 1) for g in range(ng)]

        def compute(i, b):
            def feat(f, accs):
                fi = jnp.full((16,), f, dtype=jnp.int32)
                out = []
                for g in range(ng):
                    vs = plsc.load_gather(zs[b], [eidx[g], fi])
                    vd = plsc.load_gather(zd[b], [eidx[g], fi])
                    out.append(accs[g] + vs * vd)
                return tuple(out)

            z16 = jnp.zeros((16,), jnp.float32)
            accs = lax.fori_loop(0, DOUT, feat, (z16,) * ng, unroll=8)
            for g in range(ng):
                outv[i, pl.ds(g * 16, 16)] = accs[g]

        fire(0, 0)

        def grp(g, carry):
            wait(2 * g, 0)
            fire(2 * g + 1, 1)
            compute(2 * g, 0)
            wait(2 * g + 1, 1)

            @pl.when(g < NCH2 // 2 - 1)
            def _():
                fire(2 * g + 2, 0)
            compute(2 * g + 1, 1)
            return carry

        lax.fori_loop(0, NCH2 // 2, grp, None)
        pltpu.sync_copy(outv, out_h.at[w])

    return pl.kernel(
        body,
        out_type=jax.ShapeDtypeStruct((NW, NCH2, 64), jnp.float32),
        mesh=_MESH,
        compiler_params=pltpu.CompilerParams(use_tc_tiling_on_sc=False,
                                             needs_layout_passes=False),
        scratch_types=scratch)(z, ps, pd)


_B = 1000


def _tc_layer1(a2, d0, d1, x, W1l, W1r, b1):
    def body(a_ref, d0_ref, d1_ref, x_ref, wl_ref, wr_ref, b_ref, out_ref):
        recip = 1.0 / jnp.maximum(d0_ref[...] + d1_ref[...], 1.0)
        a = jnp.concatenate(
            [a_ref[qq, 0] + a_ref[qq, 1] for qq in range(2)], axis=1) * recip
        h = jnp.dot(a, wl_ref[...], preferred_element_type=jnp.float32)
        h = h + jnp.dot(x_ref[...], wr_ref[...],
                        preferred_element_type=jnp.float32)
        h = jnp.maximum(h + b_ref[...], 0.0)
        for qq in range(4):
            out_ref[qq] = h[:, qq * FW:(qq + 1) * FW]

    return pl.pallas_call(
        body,
        grid=(N // _B,),
        in_specs=[
            pl.BlockSpec((2, NC, _B, FW), lambda i: (0, 0, i, 0)),
            pl.BlockSpec((_B, 1), lambda i: (i, 0)),
            pl.BlockSpec((_B, 1), lambda i: (i, 0)),
            pl.BlockSpec((_B, DIN), lambda i: (i, 0)),
            pl.BlockSpec((DIN, DH), lambda i: (0, 0)),
            pl.BlockSpec((DIN, DH), lambda i: (0, 0)),
            pl.BlockSpec((1, DH), lambda i: (0, 0)),
        ],
        out_specs=pl.BlockSpec((4, _B, FW), lambda i: (0, i, 0)),
        out_shape=jax.ShapeDtypeStruct((4, NP, FW), jnp.float32),
    )(a2, d0, d1, x, W1l, W1r, b1.reshape(1, DH))


def _tc_layer2(a4, d0, d1, h4, W2l, W2r, b2):
    def body(a_ref, d0_ref, d1_ref, h_ref, wl_ref, wr_ref, b_ref, out_ref):
        recip = 1.0 / jnp.maximum(d0_ref[...] + d1_ref[...], 1.0)
        a = jnp.concatenate(
            [a_ref[qq, 0] + a_ref[qq, 1] for qq in range(4)], axis=1) * recip
        hm = jnp.concatenate([h_ref[qq] for qq in range(4)], axis=1)
        z = jnp.dot(a, wl_ref[...], preferred_element_type=jnp.float32)
        z = z + jnp.dot(hm, wr_ref[...], preferred_element_type=jnp.float32)
        out_ref[...] = z + b_ref[...]

    return pl.pallas_call(
        body,
        grid=(N // _B,),
        in_specs=[
            pl.BlockSpec((4, NC, _B, FW), lambda i: (0, 0, i, 0)),
            pl.BlockSpec((_B, 1), lambda i: (i, 0)),
            pl.BlockSpec((_B, 1), lambda i: (i, 0)),
            pl.BlockSpec((4, _B, FW), lambda i: (0, i, 0)),
            pl.BlockSpec((DH, DOUT), lambda i: (0, 0)),
            pl.BlockSpec((DH, DOUT), lambda i: (0, 0)),
            pl.BlockSpec((1, DOUT), lambda i: (0, 0)),
        ],
        out_specs=pl.BlockSpec((_B, DOUT), lambda i: (i, 0)),
        out_shape=jax.ShapeDtypeStruct((N, DOUT), jnp.float32),
    )(a4, d0, d1, h4, W2l, W2r, b2.reshape(1, DOUT))


def kernel(x, edge_index, pos_edge_index, W1l, W1r, b1, W2l, W2r, b2):
    src = edge_index[0]
    dst = edge_index[1]

    off2 = (jnp.arange(2, dtype=jnp.int32) * NP)[:, None]
    off4 = (jnp.arange(4, dtype=jnp.int32) * NP)[:, None]
    src2 = (src[None, :] + off2).reshape(2, NC, NS, NCH, CH)
    src4 = (src[None, :] + off4).reshape(4, NC, NS, NCH, CH)
    dst4 = dst.reshape(NC, NS, NCH, CH)

    xp = jnp.concatenate([x, jnp.zeros((NP - N, DIN), jnp.float32)])
    x2 = xp.reshape(NP, 2, FW).transpose(1, 0, 2).reshape(2 * NP, FW)
    zeros_h = jnp.zeros((NP, FW), jnp.float32)
    zeros1 = jnp.zeros((NP,), jnp.float32)
    ones_h = jnp.ones((CH,), jnp.float32)

    agg1, deg = _make_sc_agg(2, True)(
        x2, src2, dst4, zeros_h, zeros1, ones_h)
    d0 = deg[:NP].reshape(NP, 1)
    d1 = deg[NP:].reshape(NP, 1)
    h4 = _tc_layer1(agg1, d0, d1, x, W1l, W1r, b1)
    agg2, = _make_sc_agg(4, False)(h4.reshape(4 * NP, FW), src4, dst4, zeros_h)
    z = _tc_layer2(agg2, d0, d1, h4, W2l, W2r, b2)

    ps = pos_edge_index[0].reshape(NW, NCH2, CH2)
    pd = pos_edge_index[1].reshape(NW, NCH2, CH2)
    logits = _sc_decode(z, ps, pd)
    return logits[:, :, :CH2].reshape(E)


# speedup vs baseline: 5.2844x; 2.9917x over previous
import functools

import jax
import jax.numpy as jnp
from jax import lax
from jax.experimental import pallas as pl
from jax.experimental.pallas import tpu as pltpu
from jax.experimental.pallas import tpu_sc as plsc

N = 10000
E = 160000
DIN = 256
DH = 512
DOUT = 256

FW = 128
NC = 2
NS = 16
NW = NC * NS
NP = 10112
CH = 100
EPW = E // NW
NCH = EPW // CH
RPT = NP // NS

CH2 = 50
EPW2 = E // NW
NCH2 = EPW2 // CH2

_MESH = plsc.VectorSubcoreMesh(core_axis_name="c", subcore_axis_name="s")


def _make_sc_agg(nparts, with_deg):
    out_type = [jax.ShapeDtypeStruct((nparts, NC, NP, FW), jnp.float32)]
    if with_deg:
        out_type.append(jax.ShapeDtypeStruct((NC * NP,), jnp.float32))

    scratch = [
        pltpu.VMEM_SHARED((NP, FW), jnp.float32),
        pltpu.VMEM((NCH, CH), jnp.int32),
        pltpu.VMEM((NCH, CH), jnp.int32),
        pltpu.VMEM((CH, FW), jnp.float32),
        pltpu.VMEM((CH, FW), jnp.float32),
        pltpu.SemaphoreType.DMA,
        pltpu.SemaphoreType.DMA,
        pltpu.SemaphoreType.DMA,
        pltpu.SemaphoreType.DMA,
    ]
    if with_deg:
        scratch += [
            pltpu.VMEM_SHARED((NP,), jnp.float32),
            pltpu.VMEM((CH,), jnp.float32),
            pltpu.VMEM((RPT,), jnp.float32),
            pltpu.SemaphoreType.DMA,
            pltpu.SemaphoreType.DMA,
        ]

    def body(*refs):
        if with_deg:
            (table, srcq, dstq, zeros_h, zeros1_h, ones_h,
             agg_out, deg_out,
             acc, sidx, didx, r0, r1, g0, g1, s0, s1,
             dacc, ones_v, vbuf, d0, d1) = refs
        else:
            (table, srcq, dstq, zeros_h,
             agg_out,
             acc, sidx, didx, r0, r1, g0, g1, s0, s1) = refs

        c = lax.axis_index("c")
        s = lax.axis_index("s")
        rows = [r0, r1]
        gsem = [g0, g1]
        ssem = [s0, s1]
        if with_deg:
            dsem = [d0, d1]
            pltpu.sync_copy(ones_h, ones_v)

        pltpu.sync_copy(dstq.at[c, s], didx)

        for p in range(nparts):
            pltpu.sync_copy(zeros_h.at[pl.ds(s * RPT, RPT)],
                            acc.at[pl.ds(s * RPT, RPT)])
            if with_deg and p == 0:
                pltpu.sync_copy(zeros1_h.at[pl.ds(s * RPT, RPT)], vbuf)
                pltpu.sync_copy(vbuf, dacc.at[pl.ds(s * RPT, RPT)])
            plsc.subcore_barrier()

            pltpu.sync_copy(srcq.at[p, c, s], sidx)

            def fire_g(i, b):
                pltpu.async_copy(table.at[sidx.at[i]], rows[b], gsem[b])

            def wait_g(i, b):
                pltpu.make_async_copy(table.at[sidx.at[i]], rows[b],
                                      gsem[b]).wait()

            def fire_s(i, b):
                pltpu.async_copy(rows[b], acc.at[didx.at[i]], ssem[b],
                                 add=True)

            def wait_s(i, b):
                pltpu.make_async_copy(rows[b], acc.at[didx.at[i]],
                                      ssem[b]).wait()

            def fire_d(i, b):
                if with_deg and p == 0:
                    pltpu.async_copy(ones_v, dacc.at[didx.at[i]],
                                     dsem[b], add=True)

            def wait_d(i, b):
                if with_deg and p == 0:
                    pltpu.make_async_copy(ones_v, dacc.at[didx.at[i]],
                                          dsem[b]).wait()

            fire_g(0, 0)
            wait_g(0, 0)
            fire_s(0, 0)
            fire_d(0, 0)
            fire_g(1, 1)

            def grp(g, carry):
                for j in range(2):
                    i = 2 * g + 1 + j
                    b = (1 + j) % 2
                    wait_g(i, b)
                    fire_s(i, b)
                    fire_d(i, b)
                    wait_s(i - 1, 1 - b)
                    wait_d(i - 1, 1 - b)

                    @pl.when(i + 1 < NCH)
                    def _():
                        fire_g(i + 1, 1 - b)
                return carry

            lax.fori_loop(0, (NCH - 2) // 2, grp, None)
            i = NCH - 1
            wait_g(i, i % 2)
            fire_s(i, i % 2)
            fire_d(i, i % 2)
            wait_s(i - 1, (i - 1) % 2)
            wait_d(i - 1, (i - 1) % 2)
            wait_s(i, i % 2)
            wait_d(i, i % 2)

            plsc.subcore_barrier()

            pltpu.sync_copy(acc.at[pl.ds(s * RPT, RPT)],
                            agg_out.at[p, c, pl.ds(s * RPT, RPT)])
            if with_deg and p == 0:
                pltpu.sync_copy(dacc.at[pl.ds(s * RPT, RPT)], vbuf)
                pltpu.sync_copy(vbuf, deg_out.at[pl.ds(c * NP + s * RPT, RPT)])

    return functools.partial(
        pl.kernel, body, out_type=out_type, mesh=_MESH,
        scratch_types=scratch)()


def _sc_decode(z, ps, pd):
    scratch = [
        pltpu.VMEM((NCH2, CH2), jnp.int32),
        pltpu.VMEM((NCH2, CH2), jnp.int32),
        pltpu.VMEM((CH2, DOUT), jnp.float32),
        pltpu.VMEM((CH2, DOUT), jnp.float32),
        pltpu.VMEM((CH2, DOUT), jnp.float32),
        pltpu.VMEM((CH2, DOUT), jnp.float32),
        pltpu.VMEM((NCH2, 64), jnp.float32),
        pltpu.SemaphoreType.DMA,
        pltpu.SemaphoreType.DMA,
        pltpu.SemaphoreType.DMA,
        pltpu.SemaphoreType.DMA,
    ]

    def body(z_h, ps_h, pd_h, out_h,
             psb, pdb, zs0, zs1, zd0, zd1, outv, a0, a1, b0, b1):
        c = lax.axis_index("c")
        s = lax.axis_index("s")
        w = s * NC + c
        zs = [zs0, zs1]
        zd = [zd0, zd1]
        asem = [a0, a1]
        bsem = [b0, b1]

        pltpu.sync_copy(ps_h.at[w], psb)
        pltpu.sync_copy(pd_h.at[w], pdb)

        def fire(i, b):
            pltpu.async_copy(z_h.at[psb.at[i]], zs[b], asem[b])
            pltpu.async_copy(z_h.at[pdb.at[i]], zd[b], bsem[b])

        def wait(i, b):
            pltpu.make_async_copy(z_h.at[psb.at[i]], zs[b], asem[b]).wait()
            pltpu.make_async_copy(z_h.at[pdb.at[i]], zd[b], bsem[b]).wait()

        ng = (CH2 + 15) // 16
        iota = lax.broadcasted_iota(jnp.int32, (16,), 0)
        z16 = jnp.zeros((16,), jnp.float32)

        def compute(i, b):
            def dot(e):
                acc = zs[b][e, pl.ds(0, 16)] * zd[b][e, pl.ds(0, 16)]
                for k in range(1, DOUT // 16):
                    acc = acc + (zs[b][e, pl.ds(k * 16, 16)]
                                 * zd[b][e, pl.ds(k * 16, 16)])
                return jnp.sum(acc)

            for g in range(ng):
                lo = g * 16
                cnt = min(16, CH2 - lo)

                def edge(j, vec):
                    return jnp.where(iota == j - lo, dot(j), vec)

                vec = lax.fori_loop(lo, lo + cnt, edge, z16, unroll=2)
                outv[i, pl.ds(lo, 16)] = vec

        fire(0, 0)

        def grp(g, carry):
            wait(2 * g, 0)
            fire(2 * g + 1, 1)
            compute(2 * g, 0)
            wait(2 * g + 1, 1)

            @pl.when(g < NCH2 // 2 - 1)
            def _():
                fire(2 * g + 2, 0)
            compute(2 * g + 1, 1)
            return carry

        lax.fori_loop(0, NCH2 // 2, grp, None)
        pltpu.sync_copy(outv, out_h.at[w])

    return pl.kernel(
        body,
        out_type=jax.ShapeDtypeStruct((NW, NCH2, 64), jnp.float32),
        mesh=_MESH,
        compiler_params=pltpu.CompilerParams(use_tc_tiling_on_sc=False,
                                             needs_layout_passes=False),
        scratch_types=scratch)(z, ps, pd)


_B = 1000


def _tc_layer1(a2, d0, d1, x, W1l, W1r, b1):
    def body(a_ref, d0_ref, d1_ref, x_ref, wl_ref, wr_ref, b_ref, out_ref):
        recip = 1.0 / jnp.maximum(d0_ref[...] + d1_ref[...], 1.0)
        a = jnp.concatenate(
            [a_ref[qq, 0] + a_ref[qq, 1] for qq in range(2)], axis=1) * recip
        h = jnp.dot(a, wl_ref[...], preferred_element_type=jnp.float32)
        h = h + jnp.dot(x_ref[...], wr_ref[...],
                        preferred_element_type=jnp.float32)
        h = jnp.maximum(h + b_ref[...], 0.0)
        for qq in range(4):
            out_ref[qq] = h[:, qq * FW:(qq + 1) * FW]

    return pl.pallas_call(
        body,
        grid=(N // _B,),
        in_specs=[
            pl.BlockSpec((2, NC, _B, FW), lambda i: (0, 0, i, 0)),
            pl.BlockSpec((_B, 1), lambda i: (i, 0)),
            pl.BlockSpec((_B, 1), lambda i: (i, 0)),
            pl.BlockSpec((_B, DIN), lambda i: (i, 0)),
            pl.BlockSpec((DIN, DH), lambda i: (0, 0)),
            pl.BlockSpec((DIN, DH), lambda i: (0, 0)),
            pl.BlockSpec((1, DH), lambda i: (0, 0)),
        ],
        out_specs=pl.BlockSpec((4, _B, FW), lambda i: (0, i, 0)),
        out_shape=jax.ShapeDtypeStruct((4, NP, FW), jnp.float32),
    )(a2, d0, d1, x, W1l, W1r, b1.reshape(1, DH))


def _tc_layer2(a4, d0, d1, h4, W2l, W2r, b2):
    def body(a_ref, d0_ref, d1_ref, h_ref, wl_ref, wr_ref, b_ref, out_ref):
        recip = 1.0 / jnp.maximum(d0_ref[...] + d1_ref[...], 1.0)
        a = jnp.concatenate(
            [a_ref[qq, 0] + a_ref[qq, 1] for qq in range(4)], axis=1) * recip
        hm = jnp.concatenate([h_ref[qq] for qq in range(4)], axis=1)
        z = jnp.dot(a, wl_ref[...], preferred_element_type=jnp.float32)
        z = z + jnp.dot(hm, wr_ref[...], preferred_element_type=jnp.float32)
        out_ref[...] = z + b_ref[...]

    return pl.pallas_call(
        body,
        grid=(N // _B,),
        in_specs=[
            pl.BlockSpec((4, NC, _B, FW), lambda i: (0, 0, i, 0)),
            pl.BlockSpec((_B, 1), lambda i: (i, 0)),
            pl.BlockSpec((_B, 1), lambda i: (i, 0)),
            pl.BlockSpec((4, _B, FW), lambda i: (0, i, 0)),
            pl.BlockSpec((DH, DOUT), lambda i: (0, 0)),
            pl.BlockSpec((DH, DOUT), lambda i: (0, 0)),
            pl.BlockSpec((1, DOUT), lambda i: (0, 0)),
        ],
        out_specs=pl.BlockSpec((_B, DOUT), lambda i: (i, 0)),
        out_shape=jax.ShapeDtypeStruct((N, DOUT), jnp.float32),
    )(a4, d0, d1, h4, W2l, W2r, b2.reshape(1, DOUT))


def kernel(x, edge_index, pos_edge_index, W1l, W1r, b1, W2l, W2r, b2):
    src = edge_index[0]
    dst = edge_index[1]

    off2 = (jnp.arange(2, dtype=jnp.int32) * NP)[:, None]
    off4 = (jnp.arange(4, dtype=jnp.int32) * NP)[:, None]
    src2 = (src[None, :] + off2).reshape(2, NC, NS, NCH, CH)
    src4 = (src[None, :] + off4).reshape(4, NC, NS, NCH, CH)
    dst4 = dst.reshape(NC, NS, NCH, CH)

    xp = jnp.concatenate([x, jnp.zeros((NP - N, DIN), jnp.float32)])
    x2 = xp.reshape(NP, 2, FW).transpose(1, 0, 2).reshape(2 * NP, FW)
    zeros_h = jnp.zeros((NP, FW), jnp.float32)
    zeros1 = jnp.zeros((NP,), jnp.float32)
    ones_h = jnp.ones((CH,), jnp.float32)

    agg1, deg = _make_sc_agg(2, True)(
        x2, src2, dst4, zeros_h, zeros1, ones_h)
    d0 = deg[:NP].reshape(NP, 1)
    d1 = deg[NP:].reshape(NP, 1)
    h4 = _tc_layer1(agg1, d0, d1, x, W1l, W1r, b1)
    agg2, = _make_sc_agg(4, False)(h4.reshape(4 * NP, FW), src4, dst4, zeros_h)
    z = _tc_layer2(agg2, d0, d1, h4, W2l, W2r, b2)

    ps = pos_edge_index[0].reshape(NW, NCH2, CH2)
    pd = pos_edge_index[1].reshape(NW, NCH2, CH2)
    logits = _sc_decode(z, ps, pd)
    return logits[:, :, :CH2].reshape(E)


# speedup vs baseline: 5.5109x; 1.0429x over previous
import functools

import jax
import jax.numpy as jnp
from jax import lax
from jax.experimental import pallas as pl
from jax.experimental.pallas import tpu as pltpu
from jax.experimental.pallas import tpu_sc as plsc

N = 10000
E = 160000
DIN = 256
DH = 512
DOUT = 256

FW = 128
NC = 2
NS = 16
NW = NC * NS
NP = 10112
CH = 125
EPW = E // NW
NCH = EPW // CH
RPT = NP // NS

CH2 = 50
EPW2 = E // NW
NCH2 = EPW2 // CH2

_MESH = plsc.VectorSubcoreMesh(core_axis_name="c", subcore_axis_name="s")


def _make_sc_agg(nparts, with_deg):
    out_type = [jax.ShapeDtypeStruct((nparts, NC, NP, FW), jnp.float32)]
    if with_deg:
        out_type.append(jax.ShapeDtypeStruct((NC * NP,), jnp.float32))

    scratch = [
        pltpu.VMEM_SHARED((NP, FW), jnp.float32),
        pltpu.VMEM((NCH, CH), jnp.int32),
        pltpu.VMEM((NCH, CH), jnp.int32),
        pltpu.VMEM((CH, FW), jnp.float32),
        pltpu.VMEM((CH, FW), jnp.float32),
        pltpu.SemaphoreType.DMA,
        pltpu.SemaphoreType.DMA,
        pltpu.SemaphoreType.DMA,
        pltpu.SemaphoreType.DMA,
    ]
    if with_deg:
        scratch += [
            pltpu.VMEM_SHARED((NP,), jnp.float32),
            pltpu.VMEM((CH,), jnp.float32),
            pltpu.VMEM((RPT,), jnp.float32),
            pltpu.SemaphoreType.DMA,
            pltpu.SemaphoreType.DMA,
        ]

    def body(*refs):
        if with_deg:
            (table, srcq, dstq, zeros_h, zeros1_h, ones_h,
             agg_out, deg_out,
             acc, sidx, didx, r0, r1, g0, g1, s0, s1,
             dacc, ones_v, vbuf, d0, d1) = refs
        else:
            (table, srcq, dstq, zeros_h,
             agg_out,
             acc, sidx, didx, r0, r1, g0, g1, s0, s1) = refs

        c = lax.axis_index("c")
        s = lax.axis_index("s")
        rows = [r0, r1]
        gsem = [g0, g1]
        ssem = [s0, s1]
        if with_deg:
            dsem = [d0, d1]
            pltpu.sync_copy(ones_h, ones_v)

        pltpu.sync_copy(dstq.at[c, s], didx)

        for p in range(nparts):
            pltpu.sync_copy(zeros_h.at[pl.ds(s * RPT, RPT)],
                            acc.at[pl.ds(s * RPT, RPT)])
            if with_deg and p == 0:
                pltpu.sync_copy(zeros1_h.at[pl.ds(s * RPT, RPT)], vbuf)
                pltpu.sync_copy(vbuf, dacc.at[pl.ds(s * RPT, RPT)])
            plsc.subcore_barrier()

            pltpu.sync_copy(srcq.at[p, c, s], sidx)

            def fire_g(i, b):
                pltpu.async_copy(table.at[sidx.at[i]], rows[b], gsem[b])

            def wait_g(i, b):
                pltpu.make_async_copy(table.at[sidx.at[i]], rows[b],
                                      gsem[b]).wait()

            def fire_s(i, b):
                pltpu.async_copy(rows[b], acc.at[didx.at[i]], ssem[b],
                                 add=True)

            def wait_s(i, b):
                pltpu.make_async_copy(rows[b], acc.at[didx.at[i]],
                                      ssem[b]).wait()

            def fire_d(i, b):
                if with_deg and p == 0:
                    pltpu.async_copy(ones_v, dacc.at[didx.at[i]],
                                     dsem[b], add=True)

            def wait_d(i, b):
                if with_deg and p == 0:
                    pltpu.make_async_copy(ones_v, dacc.at[didx.at[i]],
                                          dsem[b]).wait()

            fire_g(0, 0)
            wait_g(0, 0)
            fire_s(0, 0)
            fire_d(0, 0)
            fire_g(1, 1)

            def grp(g, carry):
                for j in range(2):
                    i = 2 * g + 1 + j
                    b = (1 + j) % 2
                    wait_g(i, b)
                    fire_s(i, b)
                    fire_d(i, b)
                    wait_s(i - 1, 1 - b)
                    wait_d(i - 1, 1 - b)

                    @pl.when(i + 1 < NCH)
                    def _():
                        fire_g(i + 1, 1 - b)
                return carry

            lax.fori_loop(0, (NCH - 2) // 2, grp, None)
            i = NCH - 1
            wait_g(i, i % 2)
            fire_s(i, i % 2)
            fire_d(i, i % 2)
            wait_s(i - 1, (i - 1) % 2)
            wait_d(i - 1, (i - 1) % 2)
            wait_s(i, i % 2)
            wait_d(i, i % 2)

            plsc.subcore_barrier()

            pltpu.sync_copy(acc.at[pl.ds(s * RPT, RPT)],
                            agg_out.at[p, c, pl.ds(s * RPT, RPT)])
            if with_deg and p == 0:
                pltpu.sync_copy(dacc.at[pl.ds(s * RPT, RPT)], vbuf)
                pltpu.sync_copy(vbuf, deg_out.at[pl.ds(c * NP + s * RPT, RPT)])

    return functools.partial(
        pl.kernel, body, out_type=out_type, mesh=_MESH,
        scratch_types=scratch)()


def _sc_decode(z, ps, pd):
    scratch = [
        pltpu.VMEM((NCH2, CH2), jnp.int32),
        pltpu.VMEM((NCH2, CH2), jnp.int32),
        pltpu.VMEM((CH2, DOUT), jnp.float32),
        pltpu.VMEM((CH2, DOUT), jnp.float32),
        pltpu.VMEM((CH2, DOUT), jnp.float32),
        pltpu.VMEM((CH2, DOUT), jnp.float32),
        pltpu.VMEM((NCH2, 64), jnp.float32),
        pltpu.SemaphoreType.DMA,
        pltpu.SemaphoreType.DMA,
        pltpu.SemaphoreType.DMA,
        pltpu.SemaphoreType.DMA,
    ]

    def body(z_h, ps_h, pd_h, out_h,
             psb, pdb, zs0, zs1, zd0, zd1, outv, a0, a1, b0, b1):
        c = lax.axis_index("c")
        s = lax.axis_index("s")
        w = s * NC + c
        zs = [zs0, zs1]
        zd = [zd0, zd1]
        asem = [a0, a1]
        bsem = [b0, b1]

        pltpu.sync_copy(ps_h.at[w], psb)
        pltpu.sync_copy(pd_h.at[w], pdb)

        def fire(i, b):
            pltpu.async_copy(z_h.at[psb.at[i]], zs[b], asem[b])
            pltpu.async_copy(z_h.at[pdb.at[i]], zd[b], bsem[b])

        def wait(i, b):
            pltpu.make_async_copy(z_h.at[psb.at[i]], zs[b], asem[b]).wait()
            pltpu.make_async_copy(z_h.at[pdb.at[i]], zd[b], bsem[b]).wait()

        ng = (CH2 + 15) // 16
        iota = lax.broadcasted_iota(jnp.int32, (16,), 0)
        z16 = jnp.zeros((16,), jnp.float32)

        def compute(i, b):
            def dot(e):
                acc = zs[b][e, pl.ds(0, 16)] * zd[b][e, pl.ds(0, 16)]
                for k in range(1, DOUT // 16):
                    acc = acc + (zs[b][e, pl.ds(k * 16, 16)]
                                 * zd[b][e, pl.ds(k * 16, 16)])
                return jnp.sum(acc)

            for g in range(ng):
                lo = g * 16
                cnt = min(16, CH2 - lo)

                def edge(j, vec):
                    return jnp.where(iota == j - lo, dot(j), vec)

                vec = lax.fori_loop(lo, lo + cnt, edge, z16, unroll=2)
                outv[i, pl.ds(lo, 16)] = vec

        fire(0, 0)

        def grp(g, carry):
            wait(2 * g, 0)
            fire(2 * g + 1, 1)
            compute(2 * g, 0)
            wait(2 * g + 1, 1)

            @pl.when(g < NCH2 // 2 - 1)
            def _():
                fire(2 * g + 2, 0)
            compute(2 * g + 1, 1)
            return carry

        lax.fori_loop(0, NCH2 // 2, grp, None)
        pltpu.sync_copy(outv, out_h.at[w])

    return pl.kernel(
        body,
        out_type=jax.ShapeDtypeStruct((NW, NCH2, 64), jnp.float32),
        mesh=_MESH,
        compiler_params=pltpu.CompilerParams(use_tc_tiling_on_sc=False,
                                             needs_layout_passes=False),
        scratch_types=scratch)(z, ps, pd)


_B = 1000


def _tc_lin_x(x, W1r, b1):
    def body(x_ref, w_ref, b_ref, out_ref):
        out_ref[...] = jnp.dot(x_ref[...], w_ref[...],
                               preferred_element_type=jnp.float32) + b_ref[...]

    return pl.pallas_call(
        body,
        grid=(N // _B,),
        in_specs=[
            pl.BlockSpec((_B, DIN), lambda i: (i, 0)),
            pl.BlockSpec((DIN, DH), lambda i: (0, 0)),
            pl.BlockSpec((1, DH), lambda i: (0, 0)),
        ],
        out_specs=pl.BlockSpec((_B, DH), lambda i: (i, 0)),
        out_shape=jax.ShapeDtypeStruct((N, DH), jnp.float32),
    )(x, W1r, b1.reshape(1, DH))


def _tc_combine1(a2, d0, d1, xr, W1l):
    def body(a_ref, d0_ref, d1_ref, xr_ref, wl_ref, out_ref):
        recip = 1.0 / jnp.maximum(d0_ref[...] + d1_ref[...], 1.0)
        a = jnp.concatenate(
            [a_ref[qq, 0] + a_ref[qq, 1] for qq in range(2)], axis=1) * recip
        h = jnp.dot(a, wl_ref[...], preferred_element_type=jnp.float32)
        h = jnp.maximum(h + xr_ref[...], 0.0)
        for qq in range(4):
            out_ref[qq] = h[:, qq * FW:(qq + 1) * FW]

    return pl.pallas_call(
        body,
        grid=(N // _B,),
        in_specs=[
            pl.BlockSpec((2, NC, _B, FW), lambda i: (0, 0, i, 0)),
            pl.BlockSpec((_B, 1), lambda i: (i, 0)),
            pl.BlockSpec((_B, 1), lambda i: (i, 0)),
            pl.BlockSpec((_B, DH), lambda i: (i, 0)),
            pl.BlockSpec((DIN, DH), lambda i: (0, 0)),
        ],
        out_specs=pl.BlockSpec((4, _B, FW), lambda i: (0, i, 0)),
        out_shape=jax.ShapeDtypeStruct((4, NP, FW), jnp.float32),
    )(a2, d0, d1, xr, W1l)


def _tc_lin_h(h4, W2r, b2):
    def body(h_ref, w_ref, b_ref, out_ref):
        hm = jnp.concatenate([h_ref[qq] for qq in range(4)], axis=1)
        out_ref[...] = jnp.dot(hm, w_ref[...],
                               preferred_element_type=jnp.float32) + b_ref[...]

    return pl.pallas_call(
        body,
        grid=(N // _B,),
        in_specs=[
            pl.BlockSpec((4, _B, FW), lambda i: (0, i, 0)),
            pl.BlockSpec((DH, DOUT), lambda i: (0, 0)),
            pl.BlockSpec((1, DOUT), lambda i: (0, 0)),
        ],
        out_specs=pl.BlockSpec((_B, DOUT), lambda i: (i, 0)),
        out_shape=jax.ShapeDtypeStruct((N, DOUT), jnp.float32),
    )(h4, W2r, b2.reshape(1, DOUT))


def _tc_combine2(a4, d0, d1, hr, W2l):
    def body(a_ref, d0_ref, d1_ref, hr_ref, wl_ref, out_ref):
        recip = 1.0 / jnp.maximum(d0_ref[...] + d1_ref[...], 1.0)
        a = jnp.concatenate(
            [a_ref[qq, 0] + a_ref[qq, 1] for qq in range(4)], axis=1) * recip
        z = jnp.dot(a, wl_ref[...], preferred_element_type=jnp.float32)
        out_ref[...] = z + hr_ref[...]

    return pl.pallas_call(
        body,
        grid=(N // _B,),
        in_specs=[
            pl.BlockSpec((4, NC, _B, FW), lambda i: (0, 0, i, 0)),
            pl.BlockSpec((_B, 1), lambda i: (i, 0)),
            pl.BlockSpec((_B, 1), lambda i: (i, 0)),
            pl.BlockSpec((_B, DOUT), lambda i: (i, 0)),
            pl.BlockSpec((DH, DOUT), lambda i: (0, 0)),
        ],
        out_specs=pl.BlockSpec((_B, DOUT), lambda i: (i, 0)),
        out_shape=jax.ShapeDtypeStruct((N, DOUT), jnp.float32),
    )(a4, d0, d1, hr, W2l)


def kernel(x, edge_index, pos_edge_index, W1l, W1r, b1, W2l, W2r, b2):
    src = edge_index[0]
    dst = edge_index[1]

    off2 = (jnp.arange(2, dtype=jnp.int32) * NP)[:, None]
    off4 = (jnp.arange(4, dtype=jnp.int32) * NP)[:, None]
    src2 = (src[None, :] + off2).reshape(2, NC, NS, NCH, CH)
    src4 = (src[None, :] + off4).reshape(4, NC, NS, NCH, CH)
    dst4 = dst.reshape(NC, NS, NCH, CH)

    xp = jnp.concatenate([x, jnp.zeros((NP - N, DIN), jnp.float32)])
    x2 = xp.reshape(NP, 2, FW).transpose(1, 0, 2).reshape(2 * NP, FW)
    zeros_h = jnp.zeros((NP, FW), jnp.float32)
    zeros1 = jnp.zeros((NP,), jnp.float32)
    ones_h = jnp.ones((CH,), jnp.float32)

    xr = _tc_lin_x(x, W1r, b1)
    agg1, deg = _make_sc_agg(2, True)(
        x2, src2, dst4, zeros_h, zeros1, ones_h)
    d0 = deg[:NP].reshape(NP, 1)
    d1 = deg[NP:].reshape(NP, 1)
    h4 = _tc_combine1(agg1, d0, d1, xr, W1l)
    hr = _tc_lin_h(h4, W2r, b2)
    agg2, = _make_sc_agg(4, False)(h4.reshape(4 * NP, FW), src4, dst4, zeros_h)
    z = _tc_combine2(agg2, d0, d1, hr, W2l)

    ps = pos_edge_index[0].reshape(NW, NCH2, CH2)
    pd = pos_edge_index[1].reshape(NW, NCH2, CH2)
    logits = _sc_decode(z, ps, pd)
    return logits[:, :, :CH2].reshape(E)


# speedup vs baseline: 6.0025x; 1.0892x over previous
import functools

import jax
import jax.numpy as jnp
from jax import lax
from jax.experimental import pallas as pl
from jax.experimental.pallas import tpu as pltpu
from jax.experimental.pallas import tpu_sc as plsc

N = 10000
E = 160000
DIN = 256
DH = 512
DOUT = 256

FW = 128
NC = 2
NS = 16
NW = NC * NS
NP = 10112
CH = 125
EPW = E // NW
NCH = EPW // CH
RPT = NP // NS

CH2 = 50
EPW2 = E // NW
NCH2 = EPW2 // CH2

_MESH = plsc.VectorSubcoreMesh(core_axis_name="c", subcore_axis_name="s")


def _make_sc_agg(nparts, with_deg):
    out_type = [jax.ShapeDtypeStruct((nparts, NC, NP, FW), jnp.float32)]
    if with_deg:
        out_type.append(jax.ShapeDtypeStruct((NC * NP,), jnp.float32))

    scratch = [
        pltpu.VMEM_SHARED((NP, FW), jnp.float32),
        pltpu.VMEM((NCH, CH), jnp.int32),
        pltpu.VMEM((NCH, CH), jnp.int32),
        pltpu.VMEM((CH, FW), jnp.float32),
        pltpu.VMEM((CH, FW), jnp.float32),
        pltpu.SemaphoreType.DMA,
        pltpu.SemaphoreType.DMA,
        pltpu.SemaphoreType.DMA,
        pltpu.SemaphoreType.DMA,
    ]
    if with_deg:
        scratch += [
            pltpu.VMEM_SHARED((NP,), jnp.float32),
            pltpu.VMEM((CH,), jnp.float32),
            pltpu.VMEM((RPT,), jnp.float32),
            pltpu.SemaphoreType.DMA,
            pltpu.SemaphoreType.DMA,
        ]

    def body(*refs):
        if with_deg:
            (table, srcq, dstq, zeros_h, zeros1_h, ones_h,
             agg_out, deg_out,
             acc, sidx, didx, r0, r1, g0, g1, s0, s1,
             dacc, ones_v, vbuf, d0, d1) = refs
        else:
            (table, srcq, dstq, zeros_h,
             agg_out,
             acc, sidx, didx, r0, r1, g0, g1, s0, s1) = refs

        c = lax.axis_index("c")
        s = lax.axis_index("s")
        rows = [r0, r1]
        gsem = [g0, g1]
        ssem = [s0, s1]
        if with_deg:
            dsem = [d0, d1]
            pltpu.sync_copy(ones_h, ones_v)

        pltpu.sync_copy(dstq.at[c, s], didx)

        for p in range(nparts):
            pltpu.sync_copy(zeros_h.at[pl.ds(s * RPT, RPT)],
                            acc.at[pl.ds(s * RPT, RPT)])
            if with_deg and p == 0:
                pltpu.sync_copy(zeros1_h.at[pl.ds(s * RPT, RPT)], vbuf)
                pltpu.sync_copy(vbuf, dacc.at[pl.ds(s * RPT, RPT)])
            plsc.subcore_barrier()

            pltpu.sync_copy(srcq.at[p, c, s], sidx)

            def fire_g(i, b):
                pltpu.async_copy(table.at[sidx.at[i]], rows[b], gsem[b])

            def wait_g(i, b):
                pltpu.make_async_copy(table.at[sidx.at[i]], rows[b],
                                      gsem[b]).wait()

            def fire_s(i, b):
                pltpu.async_copy(rows[b], acc.at[didx.at[i]], ssem[b],
                                 add=True)

            def wait_s(i, b):
                pltpu.make_async_copy(rows[b], acc.at[didx.at[i]],
                                      ssem[b]).wait()

            def fire_d(i, b):
                if with_deg and p == 0:
                    pltpu.async_copy(ones_v, dacc.at[didx.at[i]],
                                     dsem[b], add=True)

            def wait_d(i, b):
                if with_deg and p == 0:
                    pltpu.make_async_copy(ones_v, dacc.at[didx.at[i]],
                                          dsem[b]).wait()

            fire_g(0, 0)
            wait_g(0, 0)
            fire_s(0, 0)
            fire_d(0, 0)
            fire_g(1, 1)

            def grp(g, carry):
                for j in range(2):
                    i = 2 * g + 1 + j
                    b = (1 + j) % 2
                    wait_g(i, b)
                    fire_s(i, b)
                    fire_d(i, b)
                    wait_s(i - 1, 1 - b)
                    wait_d(i - 1, 1 - b)

                    @pl.when(i + 1 < NCH)
                    def _():
                        fire_g(i + 1, 1 - b)
                return carry

            lax.fori_loop(0, (NCH - 2) // 2, grp, None)
            i = NCH - 1
            wait_g(i, i % 2)
            fire_s(i, i % 2)
            fire_d(i, i % 2)
            wait_s(i - 1, (i - 1) % 2)
            wait_d(i - 1, (i - 1) % 2)
            wait_s(i, i % 2)
            wait_d(i, i % 2)

            plsc.subcore_barrier()

            pltpu.sync_copy(acc.at[pl.ds(s * RPT, RPT)],
                            agg_out.at[p, c, pl.ds(s * RPT, RPT)])
            if with_deg and p == 0:
                pltpu.sync_copy(dacc.at[pl.ds(s * RPT, RPT)], vbuf)
                pltpu.sync_copy(vbuf, deg_out.at[pl.ds(c * NP + s * RPT, RPT)])

    return functools.partial(
        pl.kernel, body, out_type=out_type, mesh=_MESH,
        scratch_types=scratch)()


def _sc_decode(z, ps, pd):
    scratch = [
        pltpu.VMEM((NCH2, CH2), jnp.int32),
        pltpu.VMEM((NCH2, CH2), jnp.int32),
        pltpu.VMEM((CH2, DOUT), jnp.bfloat16),
        pltpu.VMEM((CH2, DOUT), jnp.bfloat16),
        pltpu.VMEM((CH2, DOUT), jnp.bfloat16),
        pltpu.VMEM((CH2, DOUT), jnp.bfloat16),
        pltpu.VMEM((NCH2, 64), jnp.float32),
        pltpu.SemaphoreType.DMA,
        pltpu.SemaphoreType.DMA,
        pltpu.SemaphoreType.DMA,
        pltpu.SemaphoreType.DMA,
    ]

    def body(z_h, ps_h, pd_h, out_h,
             psb, pdb, zs0, zs1, zd0, zd1, outv, a0, a1, b0, b1):
        c = lax.axis_index("c")
        s = lax.axis_index("s")
        w = s * NC + c
        zs = [zs0, zs1]
        zd = [zd0, zd1]
        asem = [a0, a1]
        bsem = [b0, b1]

        pltpu.sync_copy(ps_h.at[w], psb)
        pltpu.sync_copy(pd_h.at[w], pdb)

        def fire(i, b):
            pltpu.async_copy(z_h.at[psb.at[i]], zs[b], asem[b])
            pltpu.async_copy(z_h.at[pdb.at[i]], zd[b], bsem[b])

        def wait(i, b):
            pltpu.make_async_copy(z_h.at[psb.at[i]], zs[b], asem[b]).wait()
            pltpu.make_async_copy(z_h.at[pdb.at[i]], zd[b], bsem[b]).wait()

        ng = (CH2 + 15) // 16
        iota = lax.broadcasted_iota(jnp.int32, (16,), 0)
        z16 = jnp.zeros((16,), jnp.float32)

        def compute(i, b):
            def dot(e):
                acc = jnp.zeros((16,), jnp.float32)
                acc2 = jnp.zeros((16,), jnp.float32)
                for k in range(DOUT // 32):
                    s0, s1 = plsc.unpack(zs[b][e, pl.ds(k * 32, 32)],
                                         format=plsc.PackFormat.INTERLEAVED,
                                         preferred_element_type=jnp.float32)
                    d0, d1 = plsc.unpack(zd[b][e, pl.ds(k * 32, 32)],
                                         format=plsc.PackFormat.INTERLEAVED,
                                         preferred_element_type=jnp.float32)
                    acc = acc + s0 * d0
                    acc2 = acc2 + s1 * d1
                return jnp.sum(acc + acc2)

            for g in range(ng):
                lo = g * 16
                cnt = min(16, CH2 - lo)

                def edge(j, vec):
                    return jnp.where(iota == j - lo, dot(j), vec)

                vec = lax.fori_loop(lo, lo + cnt, edge, z16, unroll=2)
                outv[i, pl.ds(lo, 16)] = vec

        fire(0, 0)

        def grp(g, carry):
            wait(2 * g, 0)
            fire(2 * g + 1, 1)
            compute(2 * g, 0)
            wait(2 * g + 1, 1)

            @pl.when(g < NCH2 // 2 - 1)
            def _():
                fire(2 * g + 2, 0)
            compute(2 * g + 1, 1)
            return carry

        lax.fori_loop(0, NCH2 // 2, grp, None)
        pltpu.sync_copy(outv, out_h.at[w])

    return pl.kernel(
        body,
        out_type=jax.ShapeDtypeStruct((NW, NCH2, 64), jnp.float32),
        mesh=_MESH,
        compiler_params=pltpu.CompilerParams(use_tc_tiling_on_sc=False,
                                             needs_layout_passes=False),
        scratch_types=scratch)(z, ps, pd)


_B = 1000


def _tc_lin_x(x, W1r, b1):
    def body(x_ref, w_ref, b_ref, out_ref):
        out_ref[...] = jnp.dot(x_ref[...], w_ref[...],
                               preferred_element_type=jnp.float32) + b_ref[...]

    return pl.pallas_call(
        body,
        grid=(N // _B,),
        in_specs=[
            pl.BlockSpec((_B, DIN), lambda i: (i, 0)),
            pl.BlockSpec((DIN, DH), lambda i: (0, 0)),
            pl.BlockSpec((1, DH), lambda i: (0, 0)),
        ],
        out_specs=pl.BlockSpec((_B, DH), lambda i: (i, 0)),
        out_shape=jax.ShapeDtypeStruct((N, DH), jnp.float32),
    )(x, W1r, b1.reshape(1, DH))


def _tc_combine1(a2, d0, d1, xr, W1l):
    def body(a_ref, d0_ref, d1_ref, xr_ref, wl_ref, out_ref):
        recip = 1.0 / jnp.maximum(d0_ref[...] + d1_ref[...], 1.0)
        a = jnp.concatenate(
            [a_ref[qq, 0] + a_ref[qq, 1] for qq in range(2)], axis=1) * recip
        h = jnp.dot(a, wl_ref[...], preferred_element_type=jnp.float32)
        h = jnp.maximum(h + xr_ref[...], 0.0)
        for qq in range(4):
            out_ref[qq] = h[:, qq * FW:(qq + 1) * FW]

    return pl.pallas_call(
        body,
        grid=(N // _B,),
        in_specs=[
            pl.BlockSpec((2, NC, _B, FW), lambda i: (0, 0, i, 0)),
            pl.BlockSpec((_B, 1), lambda i: (i, 0)),
            pl.BlockSpec((_B, 1), lambda i: (i, 0)),
            pl.BlockSpec((_B, DH), lambda i: (i, 0)),
            pl.BlockSpec((DIN, DH), lambda i: (0, 0)),
        ],
        out_specs=pl.BlockSpec((4, _B, FW), lambda i: (0, i, 0)),
        out_shape=jax.ShapeDtypeStruct((4, NP, FW), jnp.float32),
    )(a2, d0, d1, xr, W1l)


def _tc_lin_h(h4, W2r, b2):
    def body(h_ref, w_ref, b_ref, out_ref):
        hm = jnp.concatenate([h_ref[qq] for qq in range(4)], axis=1)
        out_ref[...] = jnp.dot(hm, w_ref[...],
                               preferred_element_type=jnp.float32) + b_ref[...]

    return pl.pallas_call(
        body,
        grid=(N // _B,),
        in_specs=[
            pl.BlockSpec((4, _B, FW), lambda i: (0, i, 0)),
            pl.BlockSpec((DH, DOUT), lambda i: (0, 0)),
            pl.BlockSpec((1, DOUT), lambda i: (0, 0)),
        ],
        out_specs=pl.BlockSpec((_B, DOUT), lambda i: (i, 0)),
        out_shape=jax.ShapeDtypeStruct((N, DOUT), jnp.float32),
    )(h4, W2r, b2.reshape(1, DOUT))


def _tc_combine2(a4, d0, d1, hr, W2l):
    def body(a_ref, d0_ref, d1_ref, hr_ref, wl_ref, out_ref):
        recip = 1.0 / jnp.maximum(d0_ref[...] + d1_ref[...], 1.0)
        a = jnp.concatenate(
            [a_ref[qq, 0] + a_ref[qq, 1] for qq in range(4)], axis=1) * recip
        z = jnp.dot(a, wl_ref[...], preferred_element_type=jnp.float32)
        out_ref[...] = (z + hr_ref[...]).astype(jnp.bfloat16)

    B2 = 2000
    return pl.pallas_call(
        body,
        grid=(N // B2,),
        in_specs=[
            pl.BlockSpec((4, NC, B2, FW), lambda i: (0, 0, i, 0)),
            pl.BlockSpec((B2, 1), lambda i: (i, 0)),
            pl.BlockSpec((B2, 1), lambda i: (i, 0)),
            pl.BlockSpec((B2, DOUT), lambda i: (i, 0)),
            pl.BlockSpec((DH, DOUT), lambda i: (0, 0)),
        ],
        out_specs=pl.BlockSpec((B2, DOUT), lambda i: (i, 0)),
        out_shape=jax.ShapeDtypeStruct((N, DOUT), jnp.bfloat16),
    )(a4, d0, d1, hr, W2l)


def kernel(x, edge_index, pos_edge_index, W1l, W1r, b1, W2l, W2r, b2):
    src = edge_index[0]
    dst = edge_index[1]

    off2 = (jnp.arange(2, dtype=jnp.int32) * NP)[:, None]
    off4 = (jnp.arange(4, dtype=jnp.int32) * NP)[:, None]
    src2 = (src[None, :] + off2).reshape(2, NC, NS, NCH, CH)
    src4 = (src[None, :] + off4).reshape(4, NC, NS, NCH, CH)
    dst4 = dst.reshape(NC, NS, NCH, CH)

    xp = jnp.concatenate([x, jnp.zeros((NP - N, DIN), jnp.float32)])
    x2 = xp.reshape(NP, 2, FW).transpose(1, 0, 2).reshape(2 * NP, FW)
    zeros_h = jnp.zeros((NP, FW), jnp.float32)
    zeros1 = jnp.zeros((NP,), jnp.float32)
    ones_h = jnp.ones((CH,), jnp.float32)

    xr = _tc_lin_x(x, W1r, b1)
    agg1, deg = _make_sc_agg(2, True)(
        x2, src2, dst4, zeros_h, zeros1, ones_h)
    d0 = deg[:NP].reshape(NP, 1)
    d1 = deg[NP:].reshape(NP, 1)
    h4 = _tc_combine1(agg1, d0, d1, xr, W1l)
    hr = _tc_lin_h(h4, W2r, b2)
    agg2, = _make_sc_agg(4, False)(h4.reshape(4 * NP, FW), src4, dst4, zeros_h)
    z = _tc_combine2(agg2, d0, d1, hr, W2l)

    ps = pos_edge_index[0].reshape(NW, NCH2, CH2)
    pd = pos_edge_index[1].reshape(NW, NCH2, CH2)
    logits = _sc_decode(z, ps, pd)
    return logits[:, :, :CH2].reshape(E)


# speedup vs baseline: 6.1832x; 1.0301x over previous
import functools

import jax
import jax.numpy as jnp
from jax import lax
from jax.experimental import pallas as pl
from jax.experimental.pallas import tpu as pltpu
from jax.experimental.pallas import tpu_sc as plsc

N = 10000
E = 160000
DIN = 256
DH = 512
DOUT = 256

FW = 128
NC = 2
NS = 16
NW = NC * NS
NP = 10112
CH = 125
EPW = E // NW
NCH = EPW // CH
RPT = NP // NS

CH2 = 125
EPW2 = E // NW
NCH2 = EPW2 // CH2

_MESH = plsc.VectorSubcoreMesh(core_axis_name="c", subcore_axis_name="s")


def _make_sc_agg(nparts, with_deg):
    out_type = [jax.ShapeDtypeStruct((nparts, NC, NP, FW), jnp.float32)]
    if with_deg:
        out_type.append(jax.ShapeDtypeStruct((NC * NP,), jnp.float32))

    scratch = [
        pltpu.VMEM_SHARED((NP, FW), jnp.float32),
        pltpu.VMEM((NCH, CH), jnp.int32),
        pltpu.VMEM((NCH, CH), jnp.int32),
        pltpu.VMEM((CH, FW), jnp.float32),
        pltpu.VMEM((CH, FW), jnp.float32),
        pltpu.SemaphoreType.DMA,
        pltpu.SemaphoreType.DMA,
        pltpu.SemaphoreType.DMA,
        pltpu.SemaphoreType.DMA,
    ]
    if with_deg:
        scratch += [
            pltpu.VMEM_SHARED((NP,), jnp.float32),
            pltpu.VMEM((CH,), jnp.float32),
            pltpu.VMEM((RPT,), jnp.float32),
            pltpu.SemaphoreType.DMA,
            pltpu.SemaphoreType.DMA,
        ]

    def body(*refs):
        if with_deg:
            (table, srcq, dstq, zeros_h, zeros1_h, ones_h,
             agg_out, deg_out,
             acc, sidx, didx, r0, r1, g0, g1, s0, s1,
             dacc, ones_v, vbuf, d0, d1) = refs
        else:
            (table, srcq, dstq, zeros_h,
             agg_out,
             acc, sidx, didx, r0, r1, g0, g1, s0, s1) = refs

        c = lax.axis_index("c")
        s = lax.axis_index("s")
        rows = [r0, r1]
        gsem = [g0, g1]
        ssem = [s0, s1]
        if with_deg:
            dsem = [d0, d1]
            pltpu.sync_copy(ones_h, ones_v)

        pltpu.sync_copy(dstq.at[c, s], didx)

        for p in range(nparts):
            pltpu.sync_copy(zeros_h.at[pl.ds(s * RPT, RPT)],
                            acc.at[pl.ds(s * RPT, RPT)])
            if with_deg and p == 0:
                pltpu.sync_copy(zeros1_h.at[pl.ds(s * RPT, RPT)], vbuf)
                pltpu.sync_copy(vbuf, dacc.at[pl.ds(s * RPT, RPT)])
            plsc.subcore_barrier()

            pltpu.sync_copy(srcq.at[p, c, s], sidx)

            def fire_g(i, b):
                pltpu.async_copy(table.at[sidx.at[i]], rows[b], gsem[b])

            def wait_g(i, b):
                pltpu.make_async_copy(table.at[sidx.at[i]], rows[b],
                                      gsem[b]).wait()

            def fire_s(i, b):
                pltpu.async_copy(rows[b], acc.at[didx.at[i]], ssem[b],
                                 add=True)

            def wait_s(i, b):
                pltpu.make_async_copy(rows[b], acc.at[didx.at[i]],
                                      ssem[b]).wait()

            def fire_d(i, b):
                if with_deg and p == 0:
                    pltpu.async_copy(ones_v, dacc.at[didx.at[i]],
                                     dsem[b], add=True)

            def wait_d(i, b):
                if with_deg and p == 0:
                    pltpu.make_async_copy(ones_v, dacc.at[didx.at[i]],
                                          dsem[b]).wait()

            fire_g(0, 0)
            wait_g(0, 0)
            fire_s(0, 0)
            fire_d(0, 0)
            fire_g(1, 1)

            def grp(g, carry):
                for j in range(2):
                    i = 2 * g + 1 + j
                    b = (1 + j) % 2
                    wait_g(i, b)
                    fire_s(i, b)
                    fire_d(i, b)
                    wait_s(i - 1, 1 - b)
                    wait_d(i - 1, 1 - b)

                    @pl.when(i + 1 < NCH)
                    def _():
                        fire_g(i + 1, 1 - b)
                return carry

            lax.fori_loop(0, (NCH - 2) // 2, grp, None)
            i = NCH - 1
            wait_g(i, i % 2)
            fire_s(i, i % 2)
            fire_d(i, i % 2)
            wait_s(i - 1, (i - 1) % 2)
            wait_d(i - 1, (i - 1) % 2)
            wait_s(i, i % 2)
            wait_d(i, i % 2)

            plsc.subcore_barrier()

            pltpu.sync_copy(acc.at[pl.ds(s * RPT, RPT)],
                            agg_out.at[p, c, pl.ds(s * RPT, RPT)])
            if with_deg and p == 0:
                pltpu.sync_copy(dacc.at[pl.ds(s * RPT, RPT)], vbuf)
                pltpu.sync_copy(vbuf, deg_out.at[pl.ds(c * NP + s * RPT, RPT)])

    return functools.partial(
        pl.kernel, body, out_type=out_type, mesh=_MESH,
        scratch_types=scratch)()


def _sc_decode(z, ps, pd):
    scratch = [
        pltpu.VMEM((NCH2, CH2), jnp.int32),
        pltpu.VMEM((NCH2, CH2), jnp.int32),
        pltpu.VMEM((CH2, DOUT), jnp.bfloat16),
        pltpu.VMEM((CH2, DOUT), jnp.bfloat16),
        pltpu.VMEM((CH2, DOUT), jnp.bfloat16),
        pltpu.VMEM((CH2, DOUT), jnp.bfloat16),
        pltpu.VMEM((NCH2, 128), jnp.float32),
        pltpu.SemaphoreType.DMA,
        pltpu.SemaphoreType.DMA,
        pltpu.SemaphoreType.DMA,
        pltpu.SemaphoreType.DMA,
    ]

    def body(z_h, ps_h, pd_h, out_h,
             psb, pdb, zs0, zs1, zd0, zd1, outv, a0, a1, b0, b1):
        c = lax.axis_index("c")
        s = lax.axis_index("s")
        w = s * NC + c
        zs = [zs0, zs1]
        zd = [zd0, zd1]
        asem = [a0, a1]
        bsem = [b0, b1]

        pltpu.sync_copy(ps_h.at[w], psb)
        pltpu.sync_copy(pd_h.at[w], pdb)

        def fire(i, b):
            pltpu.async_copy(z_h.at[psb.at[i]], zs[b], asem[b])
            pltpu.async_copy(z_h.at[pdb.at[i]], zd[b], bsem[b])

        def wait(i, b):
            pltpu.make_async_copy(z_h.at[psb.at[i]], zs[b], asem[b]).wait()
            pltpu.make_async_copy(z_h.at[pdb.at[i]], zd[b], bsem[b]).wait()

        ng = (CH2 + 15) // 16
        iota = lax.broadcasted_iota(jnp.int32, (16,), 0)
        z16 = jnp.zeros((16,), jnp.float32)

        def compute(i, b):
            def dot(e):
                acc = jnp.zeros((16,), jnp.float32)
                acc2 = jnp.zeros((16,), jnp.float32)
                for k in range(DOUT // 32):
                    s0, s1 = plsc.unpack(zs[b][e, pl.ds(k * 32, 32)],
                                         format=plsc.PackFormat.INTERLEAVED,
                                         preferred_element_type=jnp.float32)
                    d0, d1 = plsc.unpack(zd[b][e, pl.ds(k * 32, 32)],
                                         format=plsc.PackFormat.INTERLEAVED,
                                         preferred_element_type=jnp.float32)
                    acc = acc + s0 * d0
                    acc2 = acc2 + s1 * d1
                return jnp.sum(acc + acc2)

            for g in range(ng):
                lo = g * 16
                cnt = min(16, CH2 - lo)

                def edge(j, vec):
                    return jnp.where(iota == j - lo, dot(j), vec)

                vec = lax.fori_loop(lo, lo + cnt, edge, z16, unroll=2)
                outv[i, pl.ds(lo, 16)] = vec

        fire(0, 0)

        def grp(g, carry):
            wait(2 * g, 0)
            fire(2 * g + 1, 1)
            compute(2 * g, 0)
            wait(2 * g + 1, 1)

            @pl.when(g < NCH2 // 2 - 1)
            def _():
                fire(2 * g + 2, 0)
            compute(2 * g + 1, 1)
            return carry

        lax.fori_loop(0, NCH2 // 2, grp, None)
        pltpu.sync_copy(outv, out_h.at[w])

    return pl.kernel(
        body,
        out_type=jax.ShapeDtypeStruct((NW, NCH2, 128), jnp.float32),
        mesh=_MESH,
        compiler_params=pltpu.CompilerParams(use_tc_tiling_on_sc=False,
                                             needs_layout_passes=False),
        scratch_types=scratch)(z, ps, pd)


_B = 1000


def _tc_lin_x(x, W1r, b1):
    def body(x_ref, w_ref, b_ref, out_ref):
        out_ref[...] = jnp.dot(x_ref[...], w_ref[...],
                               preferred_element_type=jnp.float32) + b_ref[...]

    return pl.pallas_call(
        body,
        grid=(N // _B,),
        in_specs=[
            pl.BlockSpec((_B, DIN), lambda i: (i, 0)),
            pl.BlockSpec((DIN, DH), lambda i: (0, 0)),
            pl.BlockSpec((1, DH), lambda i: (0, 0)),
        ],
        out_specs=pl.BlockSpec((_B, DH), lambda i: (i, 0)),
        out_shape=jax.ShapeDtypeStruct((N, DH), jnp.float32),
    )(x, W1r, b1.reshape(1, DH))


def _tc_combine1(a2, d0, d1, xr, W1l):
    def body(a_ref, d0_ref, d1_ref, xr_ref, wl_ref, out_ref):
        recip = 1.0 / jnp.maximum(d0_ref[...] + d1_ref[...], 1.0)
        a = jnp.concatenate(
            [a_ref[qq, 0] + a_ref[qq, 1] for qq in range(2)], axis=1) * recip
        h = jnp.dot(a, wl_ref[...], preferred_element_type=jnp.float32)
        h = jnp.maximum(h + xr_ref[...], 0.0)
        for qq in range(4):
            out_ref[qq] = h[:, qq * FW:(qq + 1) * FW]

    return pl.pallas_call(
        body,
        grid=(N // _B,),
        in_specs=[
            pl.BlockSpec((2, NC, _B, FW), lambda i: (0, 0, i, 0)),
            pl.BlockSpec((_B, 1), lambda i: (i, 0)),
            pl.BlockSpec((_B, 1), lambda i: (i, 0)),
            pl.BlockSpec((_B, DH), lambda i: (i, 0)),
            pl.BlockSpec((DIN, DH), lambda i: (0, 0)),
        ],
        out_specs=pl.BlockSpec((4, _B, FW), lambda i: (0, i, 0)),
        out_shape=jax.ShapeDtypeStruct((4, NP, FW), jnp.float32),
    )(a2, d0, d1, xr, W1l)


def _tc_lin_h(h4, W2r, b2):
    def body(h_ref, w_ref, b_ref, out_ref):
        hm = jnp.concatenate([h_ref[qq] for qq in range(4)], axis=1)
        out_ref[...] = jnp.dot(hm, w_ref[...],
                               preferred_element_type=jnp.float32) + b_ref[...]

    return pl.pallas_call(
        body,
        grid=(N // _B,),
        in_specs=[
            pl.BlockSpec((4, _B, FW), lambda i: (0, i, 0)),
            pl.BlockSpec((DH, DOUT), lambda i: (0, 0)),
            pl.BlockSpec((1, DOUT), lambda i: (0, 0)),
        ],
        out_specs=pl.BlockSpec((_B, DOUT), lambda i: (i, 0)),
        out_shape=jax.ShapeDtypeStruct((N, DOUT), jnp.float32),
    )(h4, W2r, b2.reshape(1, DOUT))


def _tc_combine2(a4, d0, d1, hr, W2l):
    def body(a_ref, d0_ref, d1_ref, hr_ref, wl_ref, out_ref):
        recip = 1.0 / jnp.maximum(d0_ref[...] + d1_ref[...], 1.0)
        a = jnp.concatenate(
            [a_ref[qq, 0] + a_ref[qq, 1] for qq in range(4)], axis=1) * recip
        z = jnp.dot(a, wl_ref[...], preferred_element_type=jnp.float32)
        out_ref[...] = (z + hr_ref[...]).astype(jnp.bfloat16)

    B2 = 2000
    return pl.pallas_call(
        body,
        grid=(N // B2,),
        in_specs=[
            pl.BlockSpec((4, NC, B2, FW), lambda i: (0, 0, i, 0)),
            pl.BlockSpec((B2, 1), lambda i: (i, 0)),
            pl.BlockSpec((B2, 1), lambda i: (i, 0)),
            pl.BlockSpec((B2, DOUT), lambda i: (i, 0)),
            pl.BlockSpec((DH, DOUT), lambda i: (0, 0)),
        ],
        out_specs=pl.BlockSpec((B2, DOUT), lambda i: (i, 0)),
        out_shape=jax.ShapeDtypeStruct((N, DOUT), jnp.bfloat16),
    )(a4, d0, d1, hr, W2l)


def kernel(x, edge_index, pos_edge_index, W1l, W1r, b1, W2l, W2r, b2):
    src = edge_index[0]
    dst = edge_index[1]

    off2 = (jnp.arange(2, dtype=jnp.int32) * NP)[:, None]
    off4 = (jnp.arange(4, dtype=jnp.int32) * NP)[:, None]
    src2 = (src[None, :] + off2).reshape(2, NC, NS, NCH, CH)
    src4 = (src[None, :] + off4).reshape(4, NC, NS, NCH, CH)
    dst4 = dst.reshape(NC, NS, NCH, CH)

    xp = jnp.concatenate([x, jnp.zeros((NP - N, DIN), jnp.float32)])
    x2 = xp.reshape(NP, 2, FW).transpose(1, 0, 2).reshape(2 * NP, FW)
    zeros_h = jnp.zeros((NP, FW), jnp.float32)
    zeros1 = jnp.zeros((NP,), jnp.float32)
    ones_h = jnp.ones((CH,), jnp.float32)

    xr = _tc_lin_x(x, W1r, b1)
    agg1, deg = _make_sc_agg(2, True)(
        x2, src2, dst4, zeros_h, zeros1, ones_h)
    d0 = deg[:NP].reshape(NP, 1)
    d1 = deg[NP:].reshape(NP, 1)
    h4 = _tc_combine1(agg1, d0, d1, xr, W1l)
    hr = _tc_lin_h(h4, W2r, b2)
    agg2, = _make_sc_agg(4, False)(h4.reshape(4 * NP, FW), src4, dst4, zeros_h)
    z = _tc_combine2(agg2, d0, d1, hr, W2l)

    ps = pos_edge_index[0].reshape(NW, NCH2, CH2)
    pd = pos_edge_index[1].reshape(NW, NCH2, CH2)
    logits = _sc_decode(z, ps, pd)
    return logits[:, :, :CH2].reshape(E)
